# baseline (speedup 1.0000x reference)
import functools

import jax
import jax.numpy as jnp
from jax.experimental import pallas as pl
from jax.experimental.pallas import tpu as pltpu

N = 50000
E = 800000
EPS = 1e-5

_BE = 8000


def _bdot(a, w):
    return jnp.dot(a.astype(jnp.bfloat16), w.astype(jnp.bfloat16),
                   preferred_element_type=jnp.float32)


def _acc_stats(i, z, stats_ref, acc_ref, nsteps):
    @pl.when(i == 0)
    def _():
        acc_ref[...] = jnp.zeros_like(acc_ref)

    acc_ref[0:1, :] += jnp.sum(z, axis=0, keepdims=True)
    acc_ref[1:2, :] += jnp.sum(z * z, axis=0, keepdims=True)

    @pl.when(i == nsteps - 1)
    def _():
        stats_ref[...] = acc_ref[...]


def _first_pass_body(xi_ref, xj_ref, wt_ref, wb_ref, b_ref, w2_ref, b2_ref,
                     out_ref, stats_ref, acc_ref, *, second, nsteps):
    i = pl.program_id(0)
    xi = xi_ref[...]
    z = _bdot(xi, wt_ref[...]) + _bdot(xj_ref[...] - xi, wb_ref[...]) + b_ref[...]
    z = jnp.maximum(z, 0.0)
    if second:
        z = _bdot(z, w2_ref[...]) + b2_ref[...]
    out_ref[...] = z
    _acc_stats(i, z, stats_ref, acc_ref, nsteps)


@functools.partial(jax.jit, static_argnames=("second",))
def _first_pass(xi, xj, W0, b0, W2, b2, *, second=False):
    din = W0.shape[0] // 2
    dmid = W0.shape[1]
    if second:
        dout = W2.shape[1]
        w2_shape, b2_shape = (dmid, dout), (1, dout)
        W2_in, b2_in = W2, b2.reshape(1, dout)
    else:
        dout = dmid
        w2_shape, b2_shape = (1, 1), (1, 1)
        W2_in = jnp.zeros((1, 1), jnp.float32)
        b2_in = jnp.zeros((1, 1), jnp.float32)
    nsteps = E // _BE
    body = functools.partial(_first_pass_body, second=second, nsteps=nsteps)
    zero2 = lambda i: (0, 0)
    out, stats = pl.pallas_call(
        body,
        grid=(nsteps,),
        in_specs=[
            pl.BlockSpec((_BE, din), lambda i: (i, 0)),
            pl.BlockSpec((_BE, din), lambda i: (i, 0)),
            pl.BlockSpec((din, dmid), zero2),
            pl.BlockSpec((din, dmid), zero2),
            pl.BlockSpec((1, dmid), zero2),
            pl.BlockSpec(w2_shape, zero2),
            pl.BlockSpec(b2_shape, zero2),
        ],
        out_specs=[
            pl.BlockSpec((_BE, dout), lambda i: (i, 0)),
            pl.BlockSpec((2, dout), zero2),
        ],
        out_shape=[
            jax.ShapeDtypeStruct((E, dout), jnp.float32),
            jax.ShapeDtypeStruct((2, dout), jnp.float32),
        ],
        scratch_shapes=[pltpu.VMEM((2, dout), jnp.float32)],
    )(xi, xj, W0[:din], W0[din:], b0.reshape(1, dmid), W2_in, b2_in)
    return out, stats


def _mid_pass_body(a_ref, s_ref, t_ref, w_ref, b_ref, out_ref, stats_ref,
                   acc_ref, *, pre_relu, post_relu, nsteps):
    i = pl.program_id(0)
    a = a_ref[...] * s_ref[...] + t_ref[...]
    if pre_relu:
        a = jnp.maximum(a, 0.0)
    z = _bdot(a, w_ref[...]) + b_ref[...]
    if post_relu:
        z = jnp.maximum(z, 0.0)
    out_ref[...] = z
    _acc_stats(i, z, stats_ref, acc_ref, nsteps)


@functools.partial(jax.jit, static_argnames=("pre_relu", "post_relu"))
def _mid_pass(a_in, s, t, W, b, *, pre_relu=False, post_relu=True):
    din, dout = W.shape
    nsteps = E // _BE
    body = functools.partial(_mid_pass_body, pre_relu=pre_relu,
                             post_relu=post_relu, nsteps=nsteps)
    zero2 = lambda i: (0, 0)
    out, stats = pl.pallas_call(
        body,
        grid=(nsteps,),
        in_specs=[
            pl.BlockSpec((_BE, din), lambda i: (i, 0)),
            pl.BlockSpec((1, din), zero2),
            pl.BlockSpec((1, din), zero2),
            pl.BlockSpec((din, dout), zero2),
            pl.BlockSpec((1, dout), zero2),
        ],
        out_specs=[
            pl.BlockSpec((_BE, dout), lambda i: (i, 0)),
            pl.BlockSpec((2, dout), zero2),
        ],
        out_shape=[
            jax.ShapeDtypeStruct((E, dout), jnp.float32),
            jax.ShapeDtypeStruct((2, dout), jnp.float32),
        ],
        scratch_shapes=[pltpu.VMEM((2, dout), jnp.float32)],
    )(a_in, s.reshape(1, din), t.reshape(1, din), W, b.reshape(1, dout))
    return out, stats


def _fold(stats, g, b):
    m = stats[0] / E
    v = stats[1] / E - m * m
    s = g * jax.lax.rsqrt(v + EPS)
    return s, b - m * s


def _segsum(msg, dst):
    return jax.ops.segment_sum(msg, dst, num_segments=N)


def kernel(x, edge_index, params):
    p = params
    src = edge_index[0]
    dst = edge_index[1]
    cnt = _segsum(jnp.ones((E,), jnp.float32), dst)
    cnt_c = jnp.maximum(cnt, 1.0)[:, None]
    cnt = cnt[:, None]

    m = jnp.mean(x, axis=0)
    v = jnp.var(x, axis=0)
    h = (x - m) * jax.lax.rsqrt(v + EPS) * p["bn0_g"] + p["bn0_b"]

    dummy = jnp.zeros((1, 1), jnp.float32)

    for pre in ("e1", "e2"):
        xi = jnp.take(h, dst, axis=0)
        xj = jnp.take(h, src, axis=0)
        dmid = p[f"{pre}_w0"].shape[1]
        a1, st1 = _first_pass(xi, xj, p[f"{pre}_w0"], p[f"{pre}_b0"],
                              dummy, dummy[0])
        s1, t1 = _fold(st1, p[f"{pre}_g0"], p[f"{pre}_bb0"])
        a2, st2 = _mid_pass(a1, s1, t1, p[f"{pre}_w1"], p[f"{pre}_b1"])
        s2, t2 = _fold(st2, p[f"{pre}_g1"], p[f"{pre}_bb1"])
        a3, st3 = _mid_pass(a2, s2, t2, p[f"{pre}_w2"], p[f"{pre}_b2"])
        s3, t3 = _fold(st3, p[f"{pre}_g2"], p[f"{pre}_bb2"])
        h = (_segsum(a3, dst) * s3 + cnt * t3) / cnt_c

    xi = jnp.take(h, dst, axis=0)
    xj = jnp.take(h, src, axis=0)
    h2, st2 = _first_pass(xi, xj, p["d1_w0"], p["d1_b0"],
                          p["d1_w1"], p["d1_b1"], second=True)
    s2, t2 = _fold(st2, p["d1_g0"], p["d1_bb0"])
    a3, st3 = _mid_pass(h2, s2, t2, p["d1_w2"], p["d1_b2"], pre_relu=True)
    s3, t3 = _fold(st3, p["d1_g1"], p["d1_bb1"])
    h = (_segsum(a3, dst) * s3 + cnt * t3) / cnt_c

    xi = jnp.take(h, dst, axis=0)
    xj = jnp.take(h, src, axis=0)
    a1, st1 = _first_pass(xi, xj, p["d2_w0"], p["d2_b0"], dummy, dummy[0])
    s1, t1 = _fold(st1, p["d2_g0"], p["d2_bb0"])
    a2, st2 = _mid_pass(a1, s1, t1, p["d2_w1"], p["d2_b1"])
    s2, t2 = _fold(st2, p["d2_g1"], p["d2_bb1"])
    msg, _ = _mid_pass(a2, s2, t2, p["d2_w2"], p["d2_b2"], post_relu=False)
    return _segsum(msg, dst) / cnt_c
